# Initial kernel scaffold; baseline (speedup 1.0000x reference)
import jax
import jax.numpy as jnp
from jax.experimental import pallas as pl


def kernel(q, k, v, g, beta):
    raise NotImplementedError("write your pallas kernel here")


# speedup vs baseline: 4.4475x; 4.4475x over previous
import functools

import jax
import jax.numpy as jnp
from jax.experimental import pallas as pl
from jax.experimental.pallas import tpu as pltpu

_C = 64

_HI = jax.lax.Precision.HIGHEST


def _dot(a, b):
    return jax.lax.dot(a, b, precision=_HI)


def _dot_t(a, b):
    return jax.lax.dot_general(a, b, (((1,), (1,)), ((), ())), precision=_HI)


def _kda_kernel(q_ref, k_ref, v_ref, g_ref, b_ref, o_ref, s_ref):
    j = pl.program_id(1)

    @pl.when(j == 0)
    def _():
        s_ref[...] = jnp.zeros_like(s_ref)

    q = q_ref[0]
    k = k_ref[0]
    v = v_ref[0]
    g = g_ref[0]
    beta = b_ref[0]

    c = q.shape[0]
    scale = q.shape[-1] ** -0.5

    row = jax.lax.broadcasted_iota(jnp.int32, (c, c), 0)
    col = jax.lax.broadcasted_iota(jnp.int32, (c, c), 1)
    tril_inc = (col <= row).astype(jnp.float32)
    strict = (col < row).astype(jnp.float32)
    eye = (col == row).astype(jnp.float32)

    lg = _dot(tril_inc, g)
    lam = jnp.exp(lg)
    lam_inv = jnp.exp(-lg)
    lam_tot = lam[c - 1]

    kd = k * lam
    ki = k * lam_inv
    qd = q * (lam * scale)

    s0 = s_ref[...]

    a_mat = _dot_t(kd, ki)
    n_mat = -(beta * a_mat) * strict
    p = eye + n_mat
    nn = n_mat
    for _ in range(5):
        nn = _dot(nn, nn)
        p = p + _dot(p, nn)

    rhs = beta * (v - _dot(kd, s0))
    u = _dot(p, rhs)

    o_ref[0] = _dot(qd, s0) + _dot(_dot_t(qd, ki) * tril_inc, u)

    w = ki * lam_tot[None, :]
    s_ref[...] = lam_tot[:, None] * s0 + jax.lax.dot_general(
        w, u, (((0,), (0,)), ((), ())), precision=_HI)


@functools.partial(jax.jit, static_argnames=("interpret",))
def _run(q, k, v, g, beta, interpret=False):
    B, T, H, K = q.shape
    V = v.shape[-1]
    BH = B * H
    nc = T // _C

    def to_bh(x):
        return jnp.transpose(x, (0, 2, 1, 3)).reshape(BH, T, x.shape[-1])

    qb = to_bh(q)
    kb = to_bh(k)
    vb = to_bh(v)
    gb = to_bh(g)
    bb = jnp.transpose(beta, (0, 2, 1)).reshape(BH, T, 1)

    spec_k = pl.BlockSpec((1, _C, K), lambda i, j: (i, j, 0))
    spec_v = pl.BlockSpec((1, _C, V), lambda i, j: (i, j, 0))
    spec_b = pl.BlockSpec((1, _C, 1), lambda i, j: (i, j, 0))

    out = pl.pallas_call(
        _kda_kernel,
        out_shape=jax.ShapeDtypeStruct((BH, T, V), jnp.float32),
        grid=(BH, nc),
        in_specs=[spec_k, spec_k, spec_v, spec_k, spec_b],
        out_specs=spec_v,
        scratch_shapes=[pltpu.VMEM((K, V), jnp.float32)],
        compiler_params=pltpu.CompilerParams(
            dimension_semantics=("parallel", "arbitrary"),
        ),
        name="kda_chunked",
        interpret=interpret,
    )(qb, kb, vb, gb, bb)

    return jnp.transpose(out.reshape(B, H, T, V), (0, 2, 1, 3))


def kernel(q, k, v, g, beta):
    return _run(q, k, v, g, beta)


# speedup vs baseline: 5.6124x; 1.2619x over previous
import functools

import jax
import jax.numpy as jnp
from jax.experimental import pallas as pl
from jax.experimental.pallas import tpu as pltpu

_C = 64

def _split(a):
    hi = a.astype(jnp.bfloat16)
    lo = (a - hi.astype(jnp.float32)).astype(jnp.bfloat16)
    return hi, lo


def _dot3(a, b, dims):
    ah, al = _split(a)
    bh, bl = _split(b)

    def d(x, y):
        return jax.lax.dot_general(x, y, dims,
                                   preferred_element_type=jnp.float32)

    return d(ah, bh) + d(ah, bl) + d(al, bh)


_NN = (((1,), (0,)), ((), ()))
_NT = (((1,), (1,)), ((), ()))
_TN = (((0,), (0,)), ((), ()))


def _dot(a, b):
    return _dot3(a, b, _NN)


def _dot_t(a, b):
    return _dot3(a, b, _NT)


def _kda_kernel(q_ref, k_ref, v_ref, g_ref, b_ref, o_ref, s_ref):
    j = pl.program_id(1)

    @pl.when(j == 0)
    def _():
        s_ref[...] = jnp.zeros_like(s_ref)

    q = q_ref[0]
    k = k_ref[0]
    v = v_ref[0]
    g = g_ref[0]
    beta = b_ref[0]

    c = q.shape[0]
    scale = q.shape[-1] ** -0.5

    row = jax.lax.broadcasted_iota(jnp.int32, (c, c), 0)
    col = jax.lax.broadcasted_iota(jnp.int32, (c, c), 1)
    tril_inc = (col <= row).astype(jnp.float32)
    strict = (col < row).astype(jnp.float32)
    eye = (col == row).astype(jnp.float32)

    lg = _dot(tril_inc, g)
    lam = jnp.exp(lg)
    lam_inv = jnp.exp(-lg)
    lam_tot = lam[c - 1]

    kd = k * lam
    ki = k * lam_inv
    qd = q * (lam * scale)

    s0 = s_ref[...]

    a_mat = _dot_t(kd, ki)
    n_mat = -(beta * a_mat) * strict
    p = eye + n_mat
    nn = n_mat
    for _ in range(5):
        nn = _dot(nn, nn)
        p = p + _dot(p, nn)

    rhs = beta * (v - _dot(kd, s0))
    u = _dot(p, rhs)

    o_ref[0] = _dot(qd, s0) + _dot(_dot_t(qd, ki) * tril_inc, u)

    w = ki * lam_tot[None, :]
    s_ref[...] = lam_tot[:, None] * s0 + _dot3(w, u, _TN)


@functools.partial(jax.jit, static_argnames=("interpret",))
def _run(q, k, v, g, beta, interpret=False):
    B, T, H, K = q.shape
    V = v.shape[-1]
    BH = B * H
    nc = T // _C

    def to_bh(x):
        return jnp.transpose(x, (0, 2, 1, 3)).reshape(BH, T, x.shape[-1])

    qb = to_bh(q)
    kb = to_bh(k)
    vb = to_bh(v)
    gb = to_bh(g)
    bb = jnp.transpose(beta, (0, 2, 1)).reshape(BH, T, 1)

    spec_k = pl.BlockSpec((1, _C, K), lambda i, j: (i, j, 0))
    spec_v = pl.BlockSpec((1, _C, V), lambda i, j: (i, j, 0))
    spec_b = pl.BlockSpec((1, _C, 1), lambda i, j: (i, j, 0))

    out = pl.pallas_call(
        _kda_kernel,
        out_shape=jax.ShapeDtypeStruct((BH, T, V), jnp.float32),
        grid=(BH, nc),
        in_specs=[spec_k, spec_k, spec_v, spec_k, spec_b],
        out_specs=spec_v,
        scratch_shapes=[pltpu.VMEM((K, V), jnp.float32)],
        compiler_params=pltpu.CompilerParams(
            dimension_semantics=("parallel", "arbitrary"),
        ),
        name="kda_chunked",
        interpret=interpret,
    )(qb, kb, vb, gb, bb)

    return jnp.transpose(out.reshape(B, H, T, V), (0, 2, 1, 3))


def kernel(q, k, v, g, beta):
    return _run(q, k, v, g, beta)


# speedup vs baseline: 6.0637x; 1.0804x over previous
import functools

import jax
import jax.numpy as jnp
from jax.experimental import pallas as pl
from jax.experimental.pallas import tpu as pltpu

_C = 64
_G = 4

def _split(a):
    hi = a.astype(jnp.bfloat16)
    lo = (a - hi.astype(jnp.float32)).astype(jnp.bfloat16)
    return hi, lo


def _dot3(a, b, dims):
    ah, al = _split(a)
    bh, bl = _split(b)

    def d(x, y):
        return jax.lax.dot_general(x, y, dims,
                                   preferred_element_type=jnp.float32)

    return d(ah, bh) + d(ah, bl) + d(al, bh)


_NN = (((1,), (0,)), ((), ()))
_NT = (((1,), (1,)), ((), ()))
_TN = (((0,), (0,)), ((), ()))


def _dot(a, b):
    return _dot3(a, b, _NN)


def _dot_t(a, b):
    return _dot3(a, b, _NT)


def _kda_kernel(q_ref, k_ref, v_ref, g_ref, b_ref, o_ref, s_ref):
    j = pl.program_id(1)

    @pl.when(j == 0)
    def _():
        s_ref[...] = jnp.zeros_like(s_ref)

    grp = q_ref.shape[0]
    c = q_ref.shape[1]
    scale = q_ref.shape[2] ** -0.5

    row = jax.lax.broadcasted_iota(jnp.int32, (c, c), 0)
    col = jax.lax.broadcasted_iota(jnp.int32, (c, c), 1)
    tril_inc = (col <= row).astype(jnp.float32)
    strict = (col < row).astype(jnp.float32)
    eye = (col == row).astype(jnp.float32)
    tril_bf = tril_inc.astype(jnp.bfloat16)

    for gi in range(grp):
        q = q_ref[gi]
        k = k_ref[gi]
        v = v_ref[gi]
        g = g_ref[gi]
        beta = b_ref[gi]

        gh, gl = _split(g)
        lg = (jax.lax.dot(tril_bf, gh, preferred_element_type=jnp.float32)
              + jax.lax.dot(tril_bf, gl, preferred_element_type=jnp.float32))
        lam = jnp.exp(lg)
        lam_inv = jnp.exp(-lg)
        lam_tot = lam[c - 1]

        kd = k * lam
        ki = k * lam_inv
        qd = q * (lam * scale)

        s0 = s_ref[gi]

        a_mat = _dot_t(kd, ki)
        n_mat = -(beta * a_mat) * strict
        p = eye + n_mat
        nn = n_mat
        for _ in range(5):
            nn = _dot(nn, nn)
            p = p + _dot(p, nn)

        rhs = beta * (v - _dot(kd, s0))
        u = _dot(p, rhs)

        o_ref[gi] = _dot(qd, s0) + _dot(_dot_t(qd, ki) * tril_inc, u)

        w = ki * lam_tot[None, :]
        s_ref[gi] = lam_tot[:, None] * s0 + _dot3(w, u, _TN)


@functools.partial(jax.jit, static_argnames=("interpret",))
def _run(q, k, v, g, beta, interpret=False):
    B, T, H, K = q.shape
    V = v.shape[-1]
    BH = B * H
    nc = T // _C

    def to_bh(x):
        return jnp.transpose(x, (0, 2, 1, 3)).reshape(BH, T, x.shape[-1])

    qb = to_bh(q)
    kb = to_bh(k)
    vb = to_bh(v)
    gb = to_bh(g)
    bb = jnp.transpose(beta, (0, 2, 1)).reshape(BH, T, 1)

    spec_k = pl.BlockSpec((_G, _C, K), lambda i, j: (i, j, 0))
    spec_v = pl.BlockSpec((_G, _C, V), lambda i, j: (i, j, 0))
    spec_b = pl.BlockSpec((_G, _C, 1), lambda i, j: (i, j, 0))

    out = pl.pallas_call(
        _kda_kernel,
        out_shape=jax.ShapeDtypeStruct((BH, T, V), jnp.float32),
        grid=(BH // _G, nc),
        in_specs=[spec_k, spec_k, spec_v, spec_k, spec_b],
        out_specs=spec_v,
        scratch_shapes=[pltpu.VMEM((_G, K, V), jnp.float32)],
        compiler_params=pltpu.CompilerParams(
            dimension_semantics=("parallel", "arbitrary"),
        ),
        name="kda_chunked",
        interpret=interpret,
    )(qb, kb, vb, gb, bb)

    return jnp.transpose(out.reshape(B, H, T, V), (0, 2, 1, 3))


def kernel(q, k, v, g, beta):
    return _run(q, k, v, g, beta)


# speedup vs baseline: 13.3911x; 2.2084x over previous
import functools

import jax
import jax.numpy as jnp
from jax.experimental import pallas as pl
from jax.experimental.pallas import tpu as pltpu

_C = 64
_G = 4

def _split(a):
    hi = a.astype(jnp.bfloat16)
    lo = (a - hi.astype(jnp.float32)).astype(jnp.bfloat16)
    return hi, lo


def _dot3(a, b, dims):
    ah, al = _split(a)
    bh, bl = _split(b)

    def d(x, y):
        return jax.lax.dot_general(x, y, dims,
                                   preferred_element_type=jnp.float32)

    return d(ah, bh) + d(ah, bl) + d(al, bh)


_NN = (((1,), (0,)), ((), ()))
_NT = (((1,), (1,)), ((), ()))
_TN = (((0,), (0,)), ((), ()))


def _dot(a, b):
    return _dot3(a, b, _NN)


def _dot_t(a, b):
    return _dot3(a, b, _NT)


def _kda_kernel(q_ref, k_ref, v_ref, g_ref, b_ref, o_ref, *s_refs):
    j = pl.program_id(1)

    @pl.when(j == 0)
    def _():
        for s_ref in s_refs:
            s_ref[...] = jnp.zeros_like(s_ref)

    grp = q_ref.shape[0]
    c = q_ref.shape[1]
    scale = q_ref.shape[2] ** -0.5
    lanes = range(grp)

    row = jax.lax.broadcasted_iota(jnp.int32, (c, c), 0)
    col = jax.lax.broadcasted_iota(jnp.int32, (c, c), 1)
    tril_inc = (col <= row).astype(jnp.float32)
    strict = (col < row).astype(jnp.float32)
    eye = (col == row).astype(jnp.float32)
    tril_bf = tril_inc.astype(jnp.bfloat16)

    beta = [b_ref[gi] for gi in lanes]
    v = [v_ref[gi] for gi in lanes]
    s0 = [s_refs[gi][...] for gi in lanes]

    lam, lam_inv, lam_tot = [], [], []
    for gi in lanes:
        gh, gl = _split(g_ref[gi])
        lg = (jax.lax.dot(tril_bf, gh, preferred_element_type=jnp.float32)
              + jax.lax.dot(tril_bf, gl, preferred_element_type=jnp.float32))
        lam.append(jnp.exp(lg))
        lam_inv.append(jnp.exp(-lg))
        lam_tot.append(lam[gi][c - 1])

    kd = [k_ref[gi] * lam[gi] for gi in lanes]
    ki = [k_ref[gi] * lam_inv[gi] for gi in lanes]
    qd = [q_ref[gi] * (lam[gi] * scale) for gi in lanes]

    a_mat = [_dot_t(kd[gi], ki[gi]) for gi in lanes]
    n = [-(beta[gi] * a_mat[gi]) * strict for gi in lanes]
    p = [eye + n[gi] for gi in lanes]
    for _ in range(5):
        n = [_dot(n[gi], n[gi]) for gi in lanes]
        p = [p[gi] + _dot(p[gi], n[gi]) for gi in lanes]

    rhs = [beta[gi] * (v[gi] - _dot(kd[gi], s0[gi])) for gi in lanes]
    u = [_dot(p[gi], rhs[gi]) for gi in lanes]

    aq = [_dot_t(qd[gi], ki[gi]) * tril_inc for gi in lanes]
    for gi in lanes:
        o_ref[gi] = _dot(qd[gi], s0[gi]) + _dot(aq[gi], u[gi])

    for gi in lanes:
        w = ki[gi] * lam_tot[gi][None, :]
        s_refs[gi][...] = (lam_tot[gi][:, None] * s0[gi]
                           + _dot3(w, u[gi], _TN))


@functools.partial(jax.jit, static_argnames=("interpret",))
def _run(q, k, v, g, beta, interpret=False):
    B, T, H, K = q.shape
    V = v.shape[-1]
    BH = B * H
    nc = T // _C

    def to_bh(x):
        return jnp.transpose(x, (0, 2, 1, 3)).reshape(BH, T, x.shape[-1])

    qb = to_bh(q)
    kb = to_bh(k)
    vb = to_bh(v)
    gb = to_bh(g)
    bb = jnp.transpose(beta, (0, 2, 1)).reshape(BH, T, 1)

    spec_k = pl.BlockSpec((_G, _C, K), lambda i, j: (i, j, 0))
    spec_v = pl.BlockSpec((_G, _C, V), lambda i, j: (i, j, 0))
    spec_b = pl.BlockSpec((_G, _C, 1), lambda i, j: (i, j, 0))

    out = pl.pallas_call(
        _kda_kernel,
        out_shape=jax.ShapeDtypeStruct((BH, T, V), jnp.float32),
        grid=(BH // _G, nc),
        in_specs=[spec_k, spec_k, spec_v, spec_k, spec_b],
        out_specs=spec_v,
        scratch_shapes=[pltpu.VMEM((K, V), jnp.float32) for _ in range(_G)],
        compiler_params=pltpu.CompilerParams(
            dimension_semantics=("parallel", "arbitrary"),
        ),
        name="kda_chunked",
        interpret=interpret,
    )(qb, kb, vb, gb, bb)

    return jnp.transpose(out.reshape(B, H, T, V), (0, 2, 1, 3))


def kernel(q, k, v, g, beta):
    return _run(q, k, v, g, beta)


# speedup vs baseline: 18.0696x; 1.3494x over previous
import functools

import jax
import jax.numpy as jnp
from jax.experimental import pallas as pl
from jax.experimental.pallas import tpu as pltpu

_C = 64
_G = 8

def _split(a):
    hi = a.astype(jnp.bfloat16)
    lo = (a - hi.astype(jnp.float32)).astype(jnp.bfloat16)
    return hi, lo


def _dot3(a, b, dims):
    ah, al = _split(a)
    bh, bl = _split(b)

    def d(x, y):
        return jax.lax.dot_general(x, y, dims,
                                   preferred_element_type=jnp.float32)

    return d(ah, bh) + d(ah, bl) + d(al, bh)


_NN = (((1,), (0,)), ((), ()))
_NT = (((1,), (1,)), ((), ()))
_TN = (((0,), (0,)), ((), ()))


def _dot(a, b):
    return _dot3(a, b, _NN)


def _dot_t(a, b):
    return _dot3(a, b, _NT)


def _kda_kernel(q_ref, k_ref, v_ref, g_ref, b_ref, o_ref, *s_refs):
    j = pl.program_id(1)

    @pl.when(j == 0)
    def _():
        for s_ref in s_refs:
            s_ref[...] = jnp.zeros_like(s_ref)

    grp = q_ref.shape[0]
    c = q_ref.shape[1]
    scale = q_ref.shape[2] ** -0.5
    lanes = range(grp)

    row = jax.lax.broadcasted_iota(jnp.int32, (c, c), 0)
    col = jax.lax.broadcasted_iota(jnp.int32, (c, c), 1)
    tril_inc = (col <= row).astype(jnp.float32)
    strict = (col < row).astype(jnp.float32)
    eye = (col == row).astype(jnp.float32)
    tril_bf = tril_inc.astype(jnp.bfloat16)

    beta = [b_ref[gi] for gi in lanes]
    v = [v_ref[gi] for gi in lanes]
    s0 = [s_refs[gi][...] for gi in lanes]

    lam, lam_inv, lam_tot = [], [], []
    for gi in lanes:
        gh, gl = _split(g_ref[gi])
        lg = (jax.lax.dot(tril_bf, gh, preferred_element_type=jnp.float32)
              + jax.lax.dot(tril_bf, gl, preferred_element_type=jnp.float32))
        lam.append(jnp.exp(lg))
        lam_inv.append(jnp.exp(-lg))
        lam_tot.append(lam[gi][c - 1])

    kd = [k_ref[gi] * lam[gi] for gi in lanes]
    ki = [k_ref[gi] * lam_inv[gi] for gi in lanes]
    qd = [q_ref[gi] * (lam[gi] * scale) for gi in lanes]

    a_mat = [_dot_t(kd[gi], ki[gi]) for gi in lanes]
    n = [-(beta[gi] * a_mat[gi]) * strict for gi in lanes]
    p = [eye + n[gi] for gi in lanes]
    for _ in range(5):
        n = [_dot(n[gi], n[gi]) for gi in lanes]
        p = [p[gi] + _dot(p[gi], n[gi]) for gi in lanes]

    rhs = [beta[gi] * (v[gi] - _dot(kd[gi], s0[gi])) for gi in lanes]
    u = [_dot(p[gi], rhs[gi]) for gi in lanes]

    aq = [_dot_t(qd[gi], ki[gi]) * tril_inc for gi in lanes]
    for gi in lanes:
        o_ref[gi] = _dot(qd[gi], s0[gi]) + _dot(aq[gi], u[gi])

    for gi in lanes:
        w = ki[gi] * lam_tot[gi][None, :]
        s_refs[gi][...] = (lam_tot[gi][:, None] * s0[gi]
                           + _dot3(w, u[gi], _TN))


@functools.partial(jax.jit, static_argnames=("interpret",))
def _run(q, k, v, g, beta, interpret=False):
    B, T, H, K = q.shape
    V = v.shape[-1]
    BH = B * H
    nc = T // _C

    def to_bh(x):
        return jnp.transpose(x, (0, 2, 1, 3)).reshape(BH, T, x.shape[-1])

    qb = to_bh(q)
    kb = to_bh(k)
    vb = to_bh(v)
    gb = to_bh(g)
    bb = jnp.transpose(beta, (0, 2, 1)).reshape(BH, T, 1)

    spec_k = pl.BlockSpec((_G, _C, K), lambda i, j: (i, j, 0))
    spec_v = pl.BlockSpec((_G, _C, V), lambda i, j: (i, j, 0))
    spec_b = pl.BlockSpec((_G, _C, 1), lambda i, j: (i, j, 0))

    out = pl.pallas_call(
        _kda_kernel,
        out_shape=jax.ShapeDtypeStruct((BH, T, V), jnp.float32),
        grid=(BH // _G, nc),
        in_specs=[spec_k, spec_k, spec_v, spec_k, spec_b],
        out_specs=spec_v,
        scratch_shapes=[pltpu.VMEM((K, V), jnp.float32) for _ in range(_G)],
        compiler_params=pltpu.CompilerParams(
            dimension_semantics=("parallel", "arbitrary"),
        ),
        name="kda_chunked",
        interpret=interpret,
    )(qb, kb, vb, gb, bb)

    return jnp.transpose(out.reshape(B, H, T, V), (0, 2, 1, 3))


def kernel(q, k, v, g, beta):
    return _run(q, k, v, g, beta)


# speedup vs baseline: 20.0299x; 1.1085x over previous
import functools

import jax
import jax.numpy as jnp
from jax.experimental import pallas as pl
from jax.experimental.pallas import tpu as pltpu

_C = 64
_G = 8

def _split(a):
    hi = a.astype(jnp.bfloat16)
    lo = (a - hi.astype(jnp.float32)).astype(jnp.bfloat16)
    return hi, lo


_NN = (((1,), (0,)), ((), ()))
_NT = (((1,), (1,)), ((), ()))
_TN = (((0,), (0,)), ((), ()))


def _dot3s(a2, b2, dims):
    ah, al = a2
    bh, bl = b2

    def d(x, y):
        return jax.lax.dot_general(x, y, dims,
                                   preferred_element_type=jnp.float32)

    return d(ah, bh) + d(ah, bl) + d(al, bh)


def _dot3(a, b, dims):
    return _dot3s(_split(a), _split(b), dims)


def _kda_kernel(q_ref, k_ref, v_ref, g_ref, b_ref, o_ref, *s_refs):
    j = pl.program_id(1)

    @pl.when(j == 0)
    def _():
        for s_ref in s_refs:
            s_ref[...] = jnp.zeros_like(s_ref)

    grp = q_ref.shape[0]
    c = q_ref.shape[1]
    scale = q_ref.shape[2] ** -0.5
    lanes = range(grp)

    row = jax.lax.broadcasted_iota(jnp.int32, (c, c), 0)
    col = jax.lax.broadcasted_iota(jnp.int32, (c, c), 1)
    tril_inc = (col <= row).astype(jnp.float32)
    strict = (col < row).astype(jnp.float32)
    eye = (col == row).astype(jnp.float32)
    tril_bf = tril_inc.astype(jnp.bfloat16)

    beta = [b_ref[gi] for gi in lanes]
    v = [v_ref[gi] for gi in lanes]
    s0 = [s_refs[gi][...] for gi in lanes]

    lam, lam_inv, lam_tot = [], [], []
    for gi in lanes:
        gh, gl = _split(g_ref[gi])
        lg = (jax.lax.dot(tril_bf, gh, preferred_element_type=jnp.float32)
              + jax.lax.dot(tril_bf, gl, preferred_element_type=jnp.float32))
        lam.append(jnp.exp(lg))
        lam_inv.append(jnp.exp(-lg))
        lam_tot.append(lam[gi][c - 1])

    kq = [jnp.concatenate(
        [k_ref[gi] * lam[gi], q_ref[gi] * (lam[gi] * scale)], 0)
        for gi in lanes]
    ki = [k_ref[gi] * lam_inv[gi] for gi in lanes]

    kq2 = [_split(kq[gi]) for gi in lanes]
    ki2 = [_split(ki[gi]) for gi in lanes]

    a2 = [_dot3s(kq2[gi], ki2[gi], _NT) for gi in lanes]
    sv = [_dot3s(kq2[gi], _split(s0[gi]), _NN) for gi in lanes]

    n = [-(beta[gi] * a2[gi][:c]) * strict for gi in lanes]
    p = [eye + n[gi] for gi in lanes]
    n2 = [_split(n[gi]) for gi in lanes]
    for _ in range(5):
        n = [_dot3s(n2[gi], n2[gi], _NN) for gi in lanes]
        n2 = [_split(n[gi]) for gi in lanes]
        p2 = [_split(p[gi]) for gi in lanes]
        p = [p[gi] + _dot3s(p2[gi], n2[gi], _NN) for gi in lanes]

    rhs = [beta[gi] * (v[gi] - sv[gi][:c]) for gi in lanes]
    u = [_dot3(p[gi], rhs[gi], _NN) for gi in lanes]
    u2 = [_split(u[gi]) for gi in lanes]

    aq = [a2[gi][c:] * tril_inc for gi in lanes]
    for gi in lanes:
        o_ref[gi] = sv[gi][c:] + _dot3(aq[gi], u[gi], _NN)

    for gi in lanes:
        s_refs[gi][...] = lam_tot[gi][:, None] * (
            s0[gi] + _dot3s(ki2[gi], u2[gi], _TN))


@functools.partial(jax.jit, static_argnames=("interpret",))
def _run(q, k, v, g, beta, interpret=False):
    B, T, H, K = q.shape
    V = v.shape[-1]
    BH = B * H
    nc = T // _C

    def to_bh(x):
        return jnp.transpose(x, (0, 2, 1, 3)).reshape(BH, T, x.shape[-1])

    qb = to_bh(q)
    kb = to_bh(k)
    vb = to_bh(v)
    gb = to_bh(g)
    bb = jnp.transpose(beta, (0, 2, 1)).reshape(BH, T, 1)

    grp = _G
    while BH % grp:
        grp //= 2
    spec_k = pl.BlockSpec((grp, _C, K), lambda i, j: (i, j, 0))
    spec_v = pl.BlockSpec((grp, _C, V), lambda i, j: (i, j, 0))
    spec_b = pl.BlockSpec((grp, _C, 1), lambda i, j: (i, j, 0))

    out = pl.pallas_call(
        _kda_kernel,
        out_shape=jax.ShapeDtypeStruct((BH, T, V), jnp.float32),
        grid=(BH // grp, nc),
        in_specs=[spec_k, spec_k, spec_v, spec_k, spec_b],
        out_specs=spec_v,
        scratch_shapes=[pltpu.VMEM((K, V), jnp.float32) for _ in range(grp)],
        compiler_params=pltpu.CompilerParams(
            dimension_semantics=("parallel", "arbitrary"),
        ),
        name="kda_chunked",
        interpret=interpret,
    )(qb, kb, vb, gb, bb)

    return jnp.transpose(out.reshape(B, H, T, V), (0, 2, 1, 3))


def kernel(q, k, v, g, beta):
    return _run(q, k, v, g, beta)
